# baseline (speedup 1.0000x reference)
import functools

import jax
import jax.numpy as jnp
from jax import lax
from jax.experimental import pallas as pl
from jax.experimental.pallas import tpu as pltpu
from jax.experimental.pallas import tpu_sc as plsc

_B = 16384
_DIM = 64
_LANES = 16

_info = plsc.get_sparse_core_info()
_NC, _NS = _info.num_cores, _info.num_subcores
_NW = _NC * _NS
_BPW = _B // _NW
_GCHUNK = 128
_NCHUNK = _BPW // _GCHUNK


def _body(target_hbm, context_hbm, table_hbm, out_hbm,
          idx_t, idx_c, u_v, v_v, out_v, sem):
    wid = lax.axis_index("s") * _NC + lax.axis_index("c")
    base = wid * _BPW

    for k in range(_NCHUNK):
        pltpu.sync_copy(target_hbm.at[pl.ds(base + k * _GCHUNK, _GCHUNK)],
                        idx_t.at[k])
        pltpu.sync_copy(context_hbm.at[pl.ds(base + k * _GCHUNK, _GCHUNK)],
                        idx_c.at[k])

    copies = []
    for k in range(_NCHUNK):
        copies.append(pltpu.async_copy(
            table_hbm.at[idx_t.at[k]], u_v.at[pl.ds(k * _GCHUNK, _GCHUNK)], sem))
        copies.append(pltpu.async_copy(
            table_hbm.at[idx_c.at[k]], v_v.at[pl.ds(k * _GCHUNK, _GCHUNK)], sem))
    for c in copies:
        c.wait()

    iota = lax.iota(jnp.int32, _LANES)

    def group(g, _):
        rows = g * _LANES + iota
        acc = jnp.zeros((_LANES,), jnp.float32)
        for j in range(_DIM):
            col = jnp.full((_LANES,), j, jnp.int32)
            ug = plsc.load_gather(u_v, [rows, col])
            vg = plsc.load_gather(v_v, [rows, col])
            acc = acc + ug * vg
        out_v[pl.ds(g * _LANES, _LANES)] = acc
        return 0

    lax.fori_loop(0, _BPW // _LANES, group, 0)

    pltpu.sync_copy(out_v, out_hbm.at[pl.ds(base, _BPW)])


@jax.jit
def kernel(target, context, embedding_weights):
    mesh = plsc.VectorSubcoreMesh(core_axis_name="c", subcore_axis_name="s")
    run = pl.kernel(
        _body,
        out_type=jax.ShapeDtypeStruct((_B,), jnp.float32),
        mesh=mesh,
        compiler_params=pltpu.CompilerParams(needs_layout_passes=False,
                                             use_tc_tiling_on_sc=False),
        scratch_types=[
            pltpu.VMEM((_NCHUNK, _GCHUNK), jnp.int32),
            pltpu.VMEM((_NCHUNK, _GCHUNK), jnp.int32),
            pltpu.VMEM((_BPW, _DIM), jnp.float32),
            pltpu.VMEM((_BPW, _DIM), jnp.float32),
            pltpu.VMEM((_BPW,), jnp.float32),
            pltpu.SemaphoreType.DMA,
        ],
    )
    return run(target.astype(jnp.int32), context.astype(jnp.int32),
               embedding_weights)
